# Initial kernel scaffold; baseline (speedup 1.0000x reference)
import jax
import jax.numpy as jnp
from jax.experimental import pallas as pl


def kernel(t, x, expert_weights, params, omegas):
    raise NotImplementedError("write your pallas kernel here")


# speedup vs baseline: 1.2885x; 1.2885x over previous
import jax
import jax.numpy as jnp
from jax.experimental import pallas as pl

_ACTS = ("relu", "tanh", "swish", "gelu")
_BLOCK_B = 1024


def _erf(v):
    s = jnp.sign(v)
    a = jnp.abs(v)
    u = 1.0 / (1.0 + 0.3275911 * a)
    poly = u * (0.254829592 + u * (-0.284496736 + u * (1.421413741
           + u * (-1.453152027 + u * 1.061405429))))
    return s * (1.0 - poly * jnp.exp(-a * a))


def _apply_act(name, h):
    if name == "relu":
        return jnp.maximum(h, 0.0)
    if name == "tanh":
        return jnp.tanh(h)
    if name == "swish":
        return h * jax.nn.sigmoid(h)
    return 0.5 * h * (1.0 + _erf(h * 0.7071067811865476))


def _ensemble_body(x_ref, ew_ref, w0_ref, b0_ref, wm_ref, bm_ref,
                   wl_ref, bl_ref, o_ref, *, depths, acts):
    x = x_ref[...]
    acc = jnp.zeros(o_ref.shape, jnp.float32)
    mid = 0
    for i in range(len(depths)):
        h = jnp.dot(x, w0_ref[i], preferred_element_type=jnp.float32)
        h = _apply_act(acts[i], h + b0_ref[i])
        for _ in range(depths[i] - 2):
            h = jnp.dot(h, wm_ref[mid], preferred_element_type=jnp.float32)
            h = _apply_act(acts[i], h + bm_ref[mid])
            mid += 1
        h = jnp.dot(h, wl_ref[i], preferred_element_type=jnp.float32)
        h = h + bl_ref[i]
        acc = acc + ew_ref[:, i:i + 1] * h
    o_ref[...] = acc


def kernel(t, x, expert_weights, params, omegas):
    import functools

    batch, state_dim = x.shape
    n_exp = len(params)
    depths = tuple(len(p) for p in params)
    acts = tuple(_ACTS[i % len(_ACTS)] for i in range(n_exp))

    tb = t[0]
    w0, b0, wm, bm, wl, bl = [], [], [], [], [], []
    for i in range(n_exp):
        layers = params[i]
        first_w = layers[0]["W"]
        tf = jnp.stack([tb, jnp.sin(omegas[i] * tb), jnp.cos(omegas[i] * tb)])
        b0.append(layers[0]["b"] + first_w[:, state_dim:] @ tf)
        w0.append(first_w[:, :state_dim].T)
        for lyr in layers[1:-1]:
            wm.append(lyr["W"].T)
            bm.append(lyr["b"])
        wl.append(layers[-1]["W"].T)
        bl.append(layers[-1]["b"])

    w0 = jnp.stack(w0)
    b0 = jnp.stack(b0)
    wm = jnp.stack(wm)
    bm = jnp.stack(bm)
    wl = jnp.stack(wl)
    bl = jnp.stack(bl)

    blk = min(_BLOCK_B, batch)
    grid = (batch // blk,)
    full3 = lambda a: pl.BlockSpec(a.shape, lambda i: (0, 0, 0))
    full2 = lambda a: pl.BlockSpec(a.shape, lambda i: (0, 0))

    body = functools.partial(_ensemble_body, depths=depths, acts=acts)
    return pl.pallas_call(
        body,
        grid=grid,
        in_specs=[
            pl.BlockSpec((blk, state_dim), lambda i: (i, 0)),
            pl.BlockSpec((blk, n_exp), lambda i: (i, 0)),
            full3(w0), full2(b0), full3(wm), full2(bm), full3(wl), full2(bl),
        ],
        out_specs=pl.BlockSpec((blk, state_dim), lambda i: (i, 0)),
        out_shape=jax.ShapeDtypeStruct((batch, state_dim), jnp.float32),
    )(x, expert_weights, w0, b0, wm, bm, wl, bl)


# speedup vs baseline: 1.7248x; 1.3386x over previous
import functools

import jax
import jax.numpy as jnp
from jax import lax
from jax.experimental import pallas as pl

_ACTS = ("relu", "tanh", "swish", "gelu")
_BLOCK_B = 1024
_DN_T = (((1,), (1,)), ((), ()))


def _erf(v):
    s = jnp.sign(v)
    a = jnp.abs(v)
    u = 1.0 / (1.0 + 0.3275911 * a)
    poly = u * (0.254829592 + u * (-0.284496736 + u * (1.421413741
           + u * (-1.453152027 + u * 1.061405429))))
    return s * (1.0 - poly * jnp.exp(-a * a))


def _apply_act(name, h):
    if name == "relu":
        return jnp.maximum(h, 0.0)
    if name == "tanh":
        return jnp.tanh(h)
    if name == "swish":
        return h * jax.nn.sigmoid(h)
    return 0.5 * h * (1.0 + _erf(h * 0.7071067811865476))


def _ensemble_body(x_ref, ew_ref, tf_ref, *wb_refs, depths, acts, state_dim):
    o_ref = wb_refs[-1]
    wb_refs = wb_refs[:-1]
    x = x_ref[...]
    acc = jnp.zeros(o_ref.shape, jnp.float32)
    k = 0
    for i in range(len(depths)):
        w0 = wb_refs[k][...]
        b0 = wb_refs[k + 1][...]
        k += 2
        tfi = tf_ref[i:i + 1, :]
        h = lax.dot_general(x, w0[:, :state_dim], _DN_T,
                            preferred_element_type=jnp.float32)
        h = h + (b0 + lax.dot_general(tfi, w0[:, state_dim:], _DN_T,
                                      preferred_element_type=jnp.float32))
        h = _apply_act(acts[i], h)
        for j in range(1, depths[i]):
            w = wb_refs[k][...]
            b = wb_refs[k + 1][...]
            k += 2
            h = lax.dot_general(h, w, _DN_T,
                                preferred_element_type=jnp.float32) + b
            if j < depths[i] - 1:
                h = _apply_act(acts[i], h)
        acc = acc + ew_ref[:, i:i + 1] * h
    o_ref[...] = acc


def kernel(t, x, expert_weights, params, omegas):
    batch, state_dim = x.shape
    n_exp = len(params)
    depths = tuple(len(p) for p in params)
    acts = tuple(_ACTS[i % len(_ACTS)] for i in range(n_exp))

    tb = t[0]
    tf = jnp.stack([jnp.broadcast_to(tb, (n_exp,)),
                    jnp.sin(omegas * tb),
                    jnp.cos(omegas * tb)], axis=1)

    wb = []
    wb_specs = []
    for layers in params:
        for lyr in layers:
            w = lyr["W"]
            b = lyr["b"].reshape(1, -1)
            wb.append(w)
            wb.append(b)
            wb_specs.append(pl.BlockSpec(w.shape, lambda i: (0, 0)))
            wb_specs.append(pl.BlockSpec(b.shape, lambda i: (0, 0)))

    blk = min(_BLOCK_B, batch)
    grid = (batch // blk,)
    body = functools.partial(_ensemble_body, depths=depths, acts=acts,
                             state_dim=state_dim)
    return pl.pallas_call(
        body,
        grid=grid,
        in_specs=[
            pl.BlockSpec((blk, state_dim), lambda i: (i, 0)),
            pl.BlockSpec((blk, n_exp), lambda i: (i, 0)),
            pl.BlockSpec(tf.shape, lambda i: (0, 0)),
        ] + wb_specs,
        out_specs=pl.BlockSpec((blk, state_dim), lambda i: (i, 0)),
        out_shape=jax.ShapeDtypeStruct((batch, state_dim), jnp.float32),
    )(x, expert_weights, tf, *wb)
